# Initial kernel scaffold; baseline (speedup 1.0000x reference)
import jax
import jax.numpy as jnp
from jax.experimental import pallas as pl


def kernel(x, edge_index_e0, edge_index_e1, W0, W1):
    raise NotImplementedError("write your pallas kernel here")


# speedup vs baseline: 4.0000x; 4.0000x over previous
import functools

import jax
import jax.numpy as jnp
from jax import lax
from jax.experimental import pallas as pl
from jax.experimental.pallas import tpu as pltpu
from jax.experimental.pallas import tpu_sc as plsc

N_NODES = 10000
N_EDGES = 320000
D = 128
NPAD = 10240
CR = NPAD // D
NC = 2
NS = 16
EPT = N_EDGES // NS
K = 128
NCH = EPT // K
KT = EPT - NCH * K
ROWS_PT = NPAD // NS


def _sc_body(x_hbm, e0_hbm, e1_hbm, s_hbm, cnt_hbm,
             src_v, dst_v, srct_v, dstt_v, rows_v, cnt_v, iota_v,
             acc_sh, cnt_sh, sem_g, sem_s, sem_i):
    s = lax.axis_index("s")

    zero16 = jnp.zeros((16,), jnp.float32)

    def _zero_local():
        def zrow(r, carry):
            def zcol(q, carry2):
                rows_v[r, pl.ds(q * 16, 16)] = zero16
                return carry2
            return lax.fori_loop(0, D // 16, zcol, carry)
        lax.fori_loop(0, K, zrow, 0)

        def zcnt(r, carry):
            def zcol(q, carry2):
                cnt_v[r, pl.ds(q * 16, 16)] = zero16
                return carry2
            return lax.fori_loop(0, D // 16, zcol, carry)
        lax.fori_loop(0, CR, zcnt, 0)

    def _zero_shared():
        def zslice(r, carry):
            pltpu.sync_copy(rows_v, acc_sh.at[pl.ds(s * ROWS_PT + r * K, K)])
            return carry
        lax.fori_loop(0, ROWS_PT // K, zslice, 0)

        @pl.when(s < CR // 8)
        def _zero_cnt():
            pltpu.sync_copy(rows_v.at[pl.ds(0, 8)], cnt_sh.at[pl.ds(s * 8, 8)])

    _zero_local()
    for q in range(5):
        iota_v[pl.ds(q * 16, 16)] = lax.iota(jnp.int32, 16) + q * 16
    _zero_shared()
    plsc.subcore_barrier()

    def _count(dst_ref, nvec):
        for q in range(nvec):
            dv = dst_ref[pl.ds(q * 16, 16)]
            occ, last = plsc.scan_count(dv)
            row = lax.shift_right_logical(dv, 7)
            col = lax.bitwise_and(dv, 127)
            plsc.addupdate_scatter(cnt_v, [row, col],
                                   occ.astype(jnp.float32), mask=last)

    for et, e_hbm in ((0, e0_hbm), (1, e1_hbm)):
        def chunk(j, carry):
            base = j * K
            pltpu.async_copy(e_hbm.at[0, s, pl.ds(base, K)], src_v, sem_i)
            pltpu.async_copy(e_hbm.at[1, s, pl.ds(base, K)], dst_v, sem_i)
            pltpu.make_async_copy(e_hbm.at[0, s, pl.ds(base, K)], src_v, sem_i).wait()
            pltpu.make_async_copy(e_hbm.at[1, s, pl.ds(base, K)], dst_v, sem_i).wait()
            pltpu.async_copy(x_hbm.at[src_v], rows_v, sem_g).wait()
            pltpu.async_copy(rows_v, acc_sh.at[dst_v], sem_s, add=True).wait()
            _count(dst_v, K // 16)
            return carry
        lax.fori_loop(0, NCH, chunk, 0)

        tb = NCH * K
        pltpu.sync_copy(e_hbm.at[0, s, pl.ds(tb, KT)], srct_v)
        pltpu.sync_copy(e_hbm.at[1, s, pl.ds(tb, KT)], dstt_v)
        pltpu.async_copy(x_hbm.at[srct_v], rows_v.at[pl.ds(0, KT)], sem_g).wait()
        pltpu.async_copy(rows_v.at[pl.ds(0, KT)], acc_sh.at[dstt_v], sem_s,
                         add=True).wait()
        _count(dstt_v, KT // 16)

        pltpu.sync_copy(cnt_v, cnt_sh.at[iota_v], add=True)
        plsc.subcore_barrier()

        pltpu.sync_copy(acc_sh.at[pl.ds(s * ROWS_PT, ROWS_PT)],
                        s_hbm.at[et, pl.ds(s * ROWS_PT, ROWS_PT)])

        @pl.when(s < CR // 8)
        def _write_cnt():
            pltpu.sync_copy(cnt_sh.at[pl.ds(s * 8, 8)],
                            cnt_hbm.at[et, pl.ds(s * 8, 8)])

        if et == 0:
            _zero_local()
            _zero_shared()
            plsc.subcore_barrier()


_sc_aggregate = functools.partial(
    pl.kernel,
    out_type=(jax.ShapeDtypeStruct((NC, NPAD, D), jnp.float32),
              jax.ShapeDtypeStruct((NC, CR, D), jnp.float32)),
    mesh=plsc.VectorSubcoreMesh(core_axis_name="c", subcore_axis_name="s",
                                num_cores=1),
    scratch_types=[
        pltpu.VMEM((K,), jnp.int32),
        pltpu.VMEM((K,), jnp.int32),
        pltpu.VMEM((KT,), jnp.int32),
        pltpu.VMEM((KT,), jnp.int32),
        pltpu.VMEM((K, D), jnp.float32),
        pltpu.VMEM((CR, D), jnp.float32),
        pltpu.VMEM((CR,), jnp.int32),
        pltpu.VMEM_SHARED((NPAD, D), jnp.float32),
        pltpu.VMEM_SHARED((CR, D), jnp.float32),
        pltpu.SemaphoreType.DMA,
        pltpu.SemaphoreType.DMA,
        pltpu.SemaphoreType.DMA,
    ],
    compiler_params=pltpu.CompilerParams(needs_layout_passes=False),
)(_sc_body)


RB = 1000


def _tc_body(a0_ref, a1_ref, c0_ref, c1_ref, w0_ref, w1_ref, o_ref):
    s0 = a0_ref[0] / jnp.maximum(c0_ref[0], 1.0)
    s1 = a1_ref[0] / jnp.maximum(c1_ref[0], 1.0)
    dn = (((1,), (1,)), ((), ()))
    o_ref[...] = (
        lax.dot_general(s0, w0_ref[...], dn, preferred_element_type=jnp.float32)
        + lax.dot_general(s1, w1_ref[...], dn, preferred_element_type=jnp.float32)
    )


def kernel(x, edge_index_e0, edge_index_e1, W0, W1):
    e0 = edge_index_e0.reshape(2, NS, EPT)
    e1 = edge_index_e1.reshape(2, NS, EPT)

    acc, cnt = _sc_aggregate(x, e0, e1)
    cntc = cnt.reshape(NC, NPAD, 1)

    h = pl.pallas_call(
        _tc_body,
        grid=(N_NODES // RB,),
        in_specs=[
            pl.BlockSpec((1, RB, D), lambda i: (0, i, 0)),
            pl.BlockSpec((1, RB, D), lambda i: (1, i, 0)),
            pl.BlockSpec((1, RB, 1), lambda i: (0, i, 0)),
            pl.BlockSpec((1, RB, 1), lambda i: (1, i, 0)),
            pl.BlockSpec((D, D), lambda i: (0, 0)),
            pl.BlockSpec((D, D), lambda i: (0, 0)),
        ],
        out_specs=pl.BlockSpec((RB, D), lambda i: (i, 0)),
        out_shape=jax.ShapeDtypeStruct((N_NODES, D), jnp.float32),
    )(acc, acc, cntc, cntc, W0, W1)
    return h


# speedup vs baseline: 6.4358x; 1.6089x over previous
import functools

import jax
import jax.numpy as jnp
from jax import lax
from jax.experimental import pallas as pl
from jax.experimental.pallas import tpu as pltpu
from jax.experimental.pallas import tpu_sc as plsc

N_NODES = 10000
N_EDGES = 320000
D = 128
NPAD = 10240
CR = NPAD // D
NC = 2
NS = 16
EPT = N_EDGES // NS
K = 128
NCH = EPT // K
KT = EPT - NCH * K
ROWS_PT = NPAD // NS


def _sc_body(x_hbm, e0_hbm, e1_hbm, s_hbm, cnt_hbm,
             sb0, sb1, sb2, sb3, db0, db1, db2, db3, srct_v, dstt_v,
             rows0, rows1, cnt_v, iota_v, acc_sh, cnt_sh,
             si0, si1, si2, si3, sg0, sg1, ss0, ss1, sem_t):
    s = lax.axis_index("s")
    srcb = (sb0, sb1, sb2, sb3)
    dstb = (db0, db1, db2, db3)
    rows = (rows0, rows1)
    semi = (si0, si1, si2, si3)
    semg = (sg0, sg1)
    sems = (ss0, ss1)

    zero16 = jnp.zeros((16,), jnp.float32)

    def _zero_local():
        for rv in rows:
            def zrow(r, carry, rv=rv):
                def zcol(q, carry2):
                    rv[r, pl.ds(q * 16, 16)] = zero16
                    return carry2
                return lax.fori_loop(0, D // 16, zcol, carry)
            lax.fori_loop(0, K, zrow, 0)

        def zcnt(r, carry):
            def zcol(q, carry2):
                cnt_v[r, pl.ds(q * 16, 16)] = zero16
                return carry2
            return lax.fori_loop(0, D // 16, zcol, carry)
        lax.fori_loop(0, CR, zcnt, 0)

    def _zero_shared():
        def zslice(r, carry):
            pltpu.sync_copy(rows0, acc_sh.at[pl.ds(s * ROWS_PT + r * K, K)])
            return carry
        lax.fori_loop(0, ROWS_PT // K, zslice, 0)

        @pl.when(s < CR // 8)
        def _zero_cnt():
            pltpu.sync_copy(rows0.at[pl.ds(0, 8)], cnt_sh.at[pl.ds(s * 8, 8)])

    _zero_local()
    for q in range(5):
        iota_v[pl.ds(q * 16, 16)] = lax.iota(jnp.int32, 16) + q * 16
    _zero_shared()
    plsc.subcore_barrier()

    def _count(dst_ref, nvec):
        for q in range(nvec):
            dv = dst_ref[pl.ds(q * 16, 16)]
            occ, last = plsc.scan_count(dv)
            row = lax.shift_right_logical(dv, 7)
            col = lax.bitwise_and(dv, 127)
            plsc.addupdate_scatter(cnt_v, [row, col],
                                   occ.astype(jnp.float32), mask=last)

    for et, e_hbm in ((0, e0_hbm), (1, e1_hbm)):
        def _issue_idx(j, slot):
            pltpu.async_copy(e_hbm.at[0, s, pl.ds(j * K, K)], srcb[slot], semi[slot])
            pltpu.async_copy(e_hbm.at[1, s, pl.ds(j * K, K)], dstb[slot], semi[slot])

        def _wait_idx(j, slot):
            pltpu.make_async_copy(e_hbm.at[0, s, pl.ds(j * K, K)], srcb[slot],
                                  semi[slot]).wait()
            pltpu.make_async_copy(e_hbm.at[1, s, pl.ds(j * K, K)], dstb[slot],
                                  semi[slot]).wait()

        _issue_idx(0, 0)
        _issue_idx(1, 1)

        def group(g, carry):
            for b in range(4):
                j = 4 * g + b
                rb = b % 2

                @pl.when(j >= 2)
                def _drain():
                    pltpu.make_async_copy(
                        rows[rb], acc_sh.at[dstb[(b + 2) % 4]], sems[rb]).wait()

                @pl.when(j < NCH - 2)
                def _prefetch():
                    _issue_idx(j + 2, (b + 2) % 4)

                _wait_idx(j, b)
                pltpu.async_copy(x_hbm.at[srcb[b]], rows[rb], semg[rb]).wait()
                pltpu.async_copy(rows[rb], acc_sh.at[dstb[b]], sems[rb], add=True)
                _count(dstb[b], K // 16)
            return carry
        lax.fori_loop(0, NCH // 4, group, 0)

        pltpu.make_async_copy(rows[0], acc_sh.at[dstb[2]], sems[0]).wait()
        pltpu.make_async_copy(rows[1], acc_sh.at[dstb[3]], sems[1]).wait()

        tb = NCH * K
        pltpu.sync_copy(e_hbm.at[0, s, pl.ds(tb, KT)], srct_v)
        pltpu.sync_copy(e_hbm.at[1, s, pl.ds(tb, KT)], dstt_v)
        pltpu.async_copy(x_hbm.at[srct_v], rows0.at[pl.ds(0, KT)], sem_t).wait()
        pltpu.async_copy(rows0.at[pl.ds(0, KT)], acc_sh.at[dstt_v], sem_t,
                         add=True).wait()
        _count(dstt_v, KT // 16)

        pltpu.sync_copy(cnt_v, cnt_sh.at[iota_v], add=True)
        plsc.subcore_barrier()

        pltpu.sync_copy(acc_sh.at[pl.ds(s * ROWS_PT, ROWS_PT)],
                        s_hbm.at[et, pl.ds(s * ROWS_PT, ROWS_PT)])

        @pl.when(s < CR // 8)
        def _write_cnt():
            pltpu.sync_copy(cnt_sh.at[pl.ds(s * 8, 8)],
                            cnt_hbm.at[et, pl.ds(s * 8, 8)])

        if et == 0:
            _zero_local()
            _zero_shared()
            plsc.subcore_barrier()


_sc_aggregate = functools.partial(
    pl.kernel,
    out_type=(jax.ShapeDtypeStruct((NC, NPAD, D), jnp.float32),
              jax.ShapeDtypeStruct((NC, CR, D), jnp.float32)),
    mesh=plsc.VectorSubcoreMesh(core_axis_name="c", subcore_axis_name="s",
                                num_cores=1),
    scratch_types=(
        [pltpu.VMEM((K,), jnp.int32)] * 4
        + [pltpu.VMEM((K,), jnp.int32)] * 4
        + [pltpu.VMEM((KT,), jnp.int32)] * 2
        + [pltpu.VMEM((K, D), jnp.float32)] * 2
        + [
            pltpu.VMEM((CR, D), jnp.float32),
            pltpu.VMEM((CR,), jnp.int32),
            pltpu.VMEM_SHARED((NPAD, D), jnp.float32),
            pltpu.VMEM_SHARED((CR, D), jnp.float32),
        ]
        + [pltpu.SemaphoreType.DMA] * 9
    ),
    compiler_params=pltpu.CompilerParams(needs_layout_passes=False),
)(_sc_body)


RB = 1000


def _tc_body(a0_ref, a1_ref, c0_ref, c1_ref, w0_ref, w1_ref, o_ref):
    s0 = a0_ref[0] / jnp.maximum(c0_ref[0], 1.0)
    s1 = a1_ref[0] / jnp.maximum(c1_ref[0], 1.0)
    dn = (((1,), (1,)), ((), ()))
    o_ref[...] = (
        lax.dot_general(s0, w0_ref[...], dn, preferred_element_type=jnp.float32)
        + lax.dot_general(s1, w1_ref[...], dn, preferred_element_type=jnp.float32)
    )


def kernel(x, edge_index_e0, edge_index_e1, W0, W1):
    e0 = edge_index_e0.reshape(2, NS, EPT)
    e1 = edge_index_e1.reshape(2, NS, EPT)

    acc, cnt = _sc_aggregate(x, e0, e1)
    cntc = cnt.reshape(NC, NPAD, 1)

    h = pl.pallas_call(
        _tc_body,
        grid=(N_NODES // RB,),
        in_specs=[
            pl.BlockSpec((1, RB, D), lambda i: (0, i, 0)),
            pl.BlockSpec((1, RB, D), lambda i: (1, i, 0)),
            pl.BlockSpec((1, RB, 1), lambda i: (0, i, 0)),
            pl.BlockSpec((1, RB, 1), lambda i: (1, i, 0)),
            pl.BlockSpec((D, D), lambda i: (0, 0)),
            pl.BlockSpec((D, D), lambda i: (0, 0)),
        ],
        out_specs=pl.BlockSpec((RB, D), lambda i: (i, 0)),
        out_shape=jax.ShapeDtypeStruct((N_NODES, D), jnp.float32),
    )(acc, acc, cntc, cntc, W0, W1)
    return h


# speedup vs baseline: 8.2558x; 1.2828x over previous
import functools

import jax
import jax.numpy as jnp
from jax import lax
from jax.experimental import pallas as pl
from jax.experimental.pallas import tpu as pltpu
from jax.experimental.pallas import tpu_sc as plsc

N_NODES = 10000
N_EDGES = 320000
D = 128
NPAD = 10240
CR = NPAD // D
NC = 2
NS = 16
EPT = N_EDGES // NS
K = 128
NCH = EPT // K
KT = EPT - NCH * K
ROWS_PT = NPAD // NS


def _sc_body(x_hbm, e0_hbm, e1_hbm, s_hbm, cnt_hbm,
             sb0, sb1, sb2, sb3, db0, db1, db2, db3, srct_v, dstt_v,
             rows0, rows1, cnt_v, iota_v, acc_sh, cnt_sh,
             si0, si1, si2, si3, sg0, sg1, ss0, ss1, sem_t):
    s = lax.axis_index("s")
    srcb = (sb0, sb1, sb2, sb3)
    dstb = (db0, db1, db2, db3)
    rows = (rows0, rows1)
    semi = (si0, si1, si2, si3)
    semg = (sg0, sg1)
    sems = (ss0, ss1)

    zero16 = jnp.zeros((16,), jnp.float32)

    def _zero_local():
        for rv in rows:
            def zrow(r, carry, rv=rv):
                def zcol(q, carry2):
                    rv[r, pl.ds(q * 16, 16)] = zero16
                    return carry2
                return lax.fori_loop(0, D // 16, zcol, carry)
            lax.fori_loop(0, K, zrow, 0)

        def zcnt(r, carry):
            def zcol(q, carry2):
                cnt_v[r, pl.ds(q * 16, 16)] = zero16
                return carry2
            return lax.fori_loop(0, D // 16, zcol, carry)
        lax.fori_loop(0, CR, zcnt, 0)

    def _zero_shared():
        def zslice(r, carry):
            pltpu.sync_copy(rows0, acc_sh.at[pl.ds(s * ROWS_PT + r * K, K)])
            return carry
        lax.fori_loop(0, ROWS_PT // K, zslice, 0)

        @pl.when(s < CR // 8)
        def _zero_cnt():
            pltpu.sync_copy(rows0.at[pl.ds(0, 8)], cnt_sh.at[pl.ds(s * 8, 8)])

    _zero_local()
    for q in range(5):
        iota_v[pl.ds(q * 16, 16)] = lax.iota(jnp.int32, 16) + q * 16
    _zero_shared()
    plsc.subcore_barrier()

    def _count(dst_ref, nvec):
        for q in range(nvec):
            dv = dst_ref[pl.ds(q * 16, 16)]
            occ, last = plsc.scan_count(dv)
            row = lax.shift_right_logical(dv, 7)
            col = lax.bitwise_and(dv, 127)
            plsc.addupdate_scatter(cnt_v, [row, col],
                                   occ.astype(jnp.float32), mask=last)

    for et, e_hbm in ((0, e0_hbm), (1, e1_hbm)):
        def _issue_idx(j, slot):
            pltpu.async_copy(e_hbm.at[0, s, pl.ds(j * K, K)], srcb[slot], semi[slot])
            pltpu.async_copy(e_hbm.at[1, s, pl.ds(j * K, K)], dstb[slot], semi[slot])

        def _wait_idx(j, slot):
            pltpu.make_async_copy(e_hbm.at[0, s, pl.ds(j * K, K)], srcb[slot],
                                  semi[slot]).wait()
            pltpu.make_async_copy(e_hbm.at[1, s, pl.ds(j * K, K)], dstb[slot],
                                  semi[slot]).wait()

        _issue_idx(0, 0)
        _issue_idx(1, 1)
        _wait_idx(0, 0)
        pltpu.async_copy(x_hbm.at[srcb[0]], rows[0], semg[0])

        def group(g, carry):
            for b in range(4):
                j = 4 * g + b
                rb = b % 2
                rb1 = (b + 1) % 2

                @pl.when(j < NCH - 1)
                def _advance():
                    _wait_idx(j + 1, (b + 1) % 4)

                    @pl.when(j >= 1)
                    def _drain_prev():
                        pltpu.make_async_copy(
                            rows[rb1], acc_sh.at[dstb[(b + 3) % 4]],
                            sems[rb1]).wait()

                    @pl.when(j < NCH - 2)
                    def _prefetch():
                        _issue_idx(j + 2, (b + 2) % 4)

                    pltpu.async_copy(x_hbm.at[srcb[(b + 1) % 4]], rows[rb1],
                                     semg[rb1])

                pltpu.make_async_copy(x_hbm.at[srcb[b]], rows[rb],
                                      semg[rb]).wait()
                pltpu.async_copy(rows[rb], acc_sh.at[dstb[b]], sems[rb], add=True)
                _count(dstb[b], K // 16)
            return carry
        lax.fori_loop(0, NCH // 4, group, 0)

        pltpu.make_async_copy(rows[0], acc_sh.at[dstb[2]], sems[0]).wait()
        pltpu.make_async_copy(rows[1], acc_sh.at[dstb[3]], sems[1]).wait()

        tb = NCH * K
        pltpu.sync_copy(e_hbm.at[0, s, pl.ds(tb, KT)], srct_v)
        pltpu.sync_copy(e_hbm.at[1, s, pl.ds(tb, KT)], dstt_v)
        pltpu.async_copy(x_hbm.at[srct_v], rows0.at[pl.ds(0, KT)], sem_t).wait()
        pltpu.async_copy(rows0.at[pl.ds(0, KT)], acc_sh.at[dstt_v], sem_t,
                         add=True).wait()
        _count(dstt_v, KT // 16)

        pltpu.sync_copy(cnt_v, cnt_sh.at[iota_v], add=True)
        plsc.subcore_barrier()

        pltpu.sync_copy(acc_sh.at[pl.ds(s * ROWS_PT, ROWS_PT)],
                        s_hbm.at[et, pl.ds(s * ROWS_PT, ROWS_PT)])

        @pl.when(s < CR // 8)
        def _write_cnt():
            pltpu.sync_copy(cnt_sh.at[pl.ds(s * 8, 8)],
                            cnt_hbm.at[et, pl.ds(s * 8, 8)])

        if et == 0:
            _zero_local()
            _zero_shared()
            plsc.subcore_barrier()


_sc_aggregate = functools.partial(
    pl.kernel,
    out_type=(jax.ShapeDtypeStruct((NC, NPAD, D), jnp.float32),
              jax.ShapeDtypeStruct((NC, CR, D), jnp.float32)),
    mesh=plsc.VectorSubcoreMesh(core_axis_name="c", subcore_axis_name="s",
                                num_cores=1),
    scratch_types=(
        [pltpu.VMEM((K,), jnp.int32)] * 4
        + [pltpu.VMEM((K,), jnp.int32)] * 4
        + [pltpu.VMEM((KT,), jnp.int32)] * 2
        + [pltpu.VMEM((K, D), jnp.float32)] * 2
        + [
            pltpu.VMEM((CR, D), jnp.float32),
            pltpu.VMEM((CR,), jnp.int32),
            pltpu.VMEM_SHARED((NPAD, D), jnp.float32),
            pltpu.VMEM_SHARED((CR, D), jnp.float32),
        ]
        + [pltpu.SemaphoreType.DMA] * 9
    ),
    compiler_params=pltpu.CompilerParams(needs_layout_passes=False),
)(_sc_body)


RB = 1000


def _tc_body(a0_ref, a1_ref, c0_ref, c1_ref, w0_ref, w1_ref, o_ref):
    s0 = a0_ref[0] / jnp.maximum(c0_ref[0], 1.0)
    s1 = a1_ref[0] / jnp.maximum(c1_ref[0], 1.0)
    dn = (((1,), (1,)), ((), ()))
    o_ref[...] = (
        lax.dot_general(s0, w0_ref[...], dn, preferred_element_type=jnp.float32)
        + lax.dot_general(s1, w1_ref[...], dn, preferred_element_type=jnp.float32)
    )


def kernel(x, edge_index_e0, edge_index_e1, W0, W1):
    e0 = edge_index_e0.reshape(2, NS, EPT)
    e1 = edge_index_e1.reshape(2, NS, EPT)

    acc, cnt = _sc_aggregate(x, e0, e1)
    cntc = cnt.reshape(NC, NPAD, 1)

    h = pl.pallas_call(
        _tc_body,
        grid=(N_NODES // RB,),
        in_specs=[
            pl.BlockSpec((1, RB, D), lambda i: (0, i, 0)),
            pl.BlockSpec((1, RB, D), lambda i: (1, i, 0)),
            pl.BlockSpec((1, RB, 1), lambda i: (0, i, 0)),
            pl.BlockSpec((1, RB, 1), lambda i: (1, i, 0)),
            pl.BlockSpec((D, D), lambda i: (0, 0)),
            pl.BlockSpec((D, D), lambda i: (0, 0)),
        ],
        out_specs=pl.BlockSpec((RB, D), lambda i: (i, 0)),
        out_shape=jax.ShapeDtypeStruct((N_NODES, D), jnp.float32),
    )(acc, acc, cntc, cntc, W0, W1)
    return h


# speedup vs baseline: 14.3712x; 1.7407x over previous
import functools

import jax
import jax.numpy as jnp
from jax import lax
from jax.experimental import pallas as pl
from jax.experimental.pallas import tpu as pltpu
from jax.experimental.pallas import tpu_sc as plsc

N_NODES = 10000
N_EDGES = 320000
D = 128
NPAD = 10240
CR = NPAD // D
NC = 2
NS = 16
EPT = N_EDGES // NS
K = 128
NCH = EPT // K
KT = EPT - NCH * K
ROWS_PT = NPAD // NS


def _sc_body(x_hbm, eidx_hbm, s_hbm, cnt_hbm,
             sb0, sb1, sb2, sb3, db0, db1, db2, db3, srct_v, dstt_v,
             rows0, rows1, cnt_v, iota_v, acc_sh, cnt_sh,
             si0, si1, si2, si3, sg0, sg1, ss0, ss1, sem_t):
    c = lax.axis_index("c")
    s = lax.axis_index("s")
    srcb = (sb0, sb1, sb2, sb3)
    dstb = (db0, db1, db2, db3)
    rows = (rows0, rows1)
    semi = (si0, si1, si2, si3)
    semg = (sg0, sg1)
    sems = (ss0, ss1)

    zero16 = jnp.zeros((16,), jnp.float32)

    def _zero_local():
        for rv in rows:
            def zrow(r, carry, rv=rv):
                def zcol(q, carry2):
                    rv[r, pl.ds(q * 16, 16)] = zero16
                    return carry2
                return lax.fori_loop(0, D // 16, zcol, carry)
            lax.fori_loop(0, K, zrow, 0)

        def zcnt(r, carry):
            def zcol(q, carry2):
                cnt_v[r, pl.ds(q * 16, 16)] = zero16
                return carry2
            return lax.fori_loop(0, D // 16, zcol, carry)
        lax.fori_loop(0, CR, zcnt, 0)

    def _zero_shared():
        def zslice(r, carry):
            pltpu.sync_copy(rows0, acc_sh.at[pl.ds(s * ROWS_PT + r * K, K)])
            return carry
        lax.fori_loop(0, ROWS_PT // K, zslice, 0)

        @pl.when(s < CR // 8)
        def _zero_cnt():
            pltpu.sync_copy(rows0.at[pl.ds(0, 8)], cnt_sh.at[pl.ds(s * 8, 8)])

    _zero_local()
    for q in range(5):
        iota_v[pl.ds(q * 16, 16)] = lax.iota(jnp.int32, 16) + q * 16
    _zero_shared()
    plsc.subcore_barrier()

    def _count(dst_ref, nvec):
        for q in range(nvec):
            dv = dst_ref[pl.ds(q * 16, 16)]
            occ, last = plsc.scan_count(dv)
            row = lax.shift_right_logical(dv, 7)
            col = lax.bitwise_and(dv, 127)
            plsc.addupdate_scatter(cnt_v, [row, col],
                                   occ.astype(jnp.float32), mask=last)

    if True:
        def _issue_idx(j, slot):
            pltpu.async_copy(eidx_hbm.at[c, 0, s, pl.ds(j * K, K)], srcb[slot], semi[slot])
            pltpu.async_copy(eidx_hbm.at[c, 1, s, pl.ds(j * K, K)], dstb[slot], semi[slot])

        def _wait_idx(j, slot):
            pltpu.make_async_copy(eidx_hbm.at[c, 0, s, pl.ds(j * K, K)], srcb[slot],
                                  semi[slot]).wait()
            pltpu.make_async_copy(eidx_hbm.at[c, 1, s, pl.ds(j * K, K)], dstb[slot],
                                  semi[slot]).wait()

        _issue_idx(0, 0)
        _issue_idx(1, 1)
        _wait_idx(0, 0)
        pltpu.async_copy(x_hbm.at[srcb[0]], rows[0], semg[0])

        def group(g, carry):
            for b in range(4):
                j = 4 * g + b
                rb = b % 2
                rb1 = (b + 1) % 2

                @pl.when(j < NCH - 1)
                def _advance():
                    _wait_idx(j + 1, (b + 1) % 4)

                    @pl.when(j >= 1)
                    def _drain_prev():
                        pltpu.make_async_copy(
                            rows[rb1], acc_sh.at[dstb[(b + 3) % 4]],
                            sems[rb1]).wait()

                    @pl.when(j < NCH - 2)
                    def _prefetch():
                        _issue_idx(j + 2, (b + 2) % 4)

                    pltpu.async_copy(x_hbm.at[srcb[(b + 1) % 4]], rows[rb1],
                                     semg[rb1])

                pltpu.make_async_copy(x_hbm.at[srcb[b]], rows[rb],
                                      semg[rb]).wait()
                pltpu.async_copy(rows[rb], acc_sh.at[dstb[b]], sems[rb], add=True)
                _count(dstb[b], K // 16)
            return carry
        lax.fori_loop(0, NCH // 4, group, 0)

        pltpu.make_async_copy(rows[0], acc_sh.at[dstb[2]], sems[0]).wait()
        pltpu.make_async_copy(rows[1], acc_sh.at[dstb[3]], sems[1]).wait()

        tb = NCH * K
        pltpu.sync_copy(eidx_hbm.at[c, 0, s, pl.ds(tb, KT)], srct_v)
        pltpu.sync_copy(eidx_hbm.at[c, 1, s, pl.ds(tb, KT)], dstt_v)
        pltpu.async_copy(x_hbm.at[srct_v], rows0.at[pl.ds(0, KT)], sem_t).wait()
        pltpu.async_copy(rows0.at[pl.ds(0, KT)], acc_sh.at[dstt_v], sem_t,
                         add=True).wait()
        _count(dstt_v, KT // 16)

        pltpu.sync_copy(cnt_v, cnt_sh.at[iota_v], add=True)
        plsc.subcore_barrier()

        pltpu.sync_copy(acc_sh.at[pl.ds(s * ROWS_PT, ROWS_PT)],
                        s_hbm.at[c, pl.ds(s * ROWS_PT, ROWS_PT)])

        @pl.when(s < CR // 8)
        def _write_cnt():
            pltpu.sync_copy(cnt_sh.at[pl.ds(s * 8, 8)],
                            cnt_hbm.at[c, pl.ds(s * 8, 8)])


_sc_aggregate = functools.partial(
    pl.kernel,
    out_type=(jax.ShapeDtypeStruct((NC, NPAD, D), jnp.float32),
              jax.ShapeDtypeStruct((NC, CR, D), jnp.float32)),
    mesh=plsc.VectorSubcoreMesh(core_axis_name="c", subcore_axis_name="s",
                                num_cores=2),
    scratch_types=(
        [pltpu.VMEM((K,), jnp.int32)] * 4
        + [pltpu.VMEM((K,), jnp.int32)] * 4
        + [pltpu.VMEM((KT,), jnp.int32)] * 2
        + [pltpu.VMEM((K, D), jnp.float32)] * 2
        + [
            pltpu.VMEM((CR, D), jnp.float32),
            pltpu.VMEM((CR,), jnp.int32),
            pltpu.VMEM_SHARED((NPAD, D), jnp.float32),
            pltpu.VMEM_SHARED((CR, D), jnp.float32),
        ]
        + [pltpu.SemaphoreType.DMA] * 9
    ),
    compiler_params=pltpu.CompilerParams(needs_layout_passes=False),
)(_sc_body)


RB = 1000


def _tc_body(a0_ref, a1_ref, c0_ref, c1_ref, w0_ref, w1_ref, o_ref):
    s0 = a0_ref[0] / jnp.maximum(c0_ref[0], 1.0)
    s1 = a1_ref[0] / jnp.maximum(c1_ref[0], 1.0)
    dn = (((1,), (1,)), ((), ()))
    o_ref[...] = (
        lax.dot_general(s0, w0_ref[...], dn, preferred_element_type=jnp.float32)
        + lax.dot_general(s1, w1_ref[...], dn, preferred_element_type=jnp.float32)
    )


def kernel(x, edge_index_e0, edge_index_e1, W0, W1):
    eidx = jnp.stack([edge_index_e0, edge_index_e1]).reshape(NC, 2, NS, EPT)

    acc, cnt = _sc_aggregate(x, eidx)
    cntc = cnt.reshape(NC, NPAD, 1)

    h = pl.pallas_call(
        _tc_body,
        grid=(N_NODES // RB,),
        in_specs=[
            pl.BlockSpec((1, RB, D), lambda i: (0, i, 0)),
            pl.BlockSpec((1, RB, D), lambda i: (1, i, 0)),
            pl.BlockSpec((1, RB, 1), lambda i: (0, i, 0)),
            pl.BlockSpec((1, RB, 1), lambda i: (1, i, 0)),
            pl.BlockSpec((D, D), lambda i: (0, 0)),
            pl.BlockSpec((D, D), lambda i: (0, 0)),
        ],
        out_specs=pl.BlockSpec((RB, D), lambda i: (i, 0)),
        out_shape=jax.ShapeDtypeStruct((N_NODES, D), jnp.float32),
    )(acc, acc, cntc, cntc, W0, W1)
    return h


# speedup vs baseline: 14.4418x; 1.0049x over previous
import functools

import jax
import jax.numpy as jnp
from jax import lax
from jax.experimental import pallas as pl
from jax.experimental.pallas import tpu as pltpu
from jax.experimental.pallas import tpu_sc as plsc

N_NODES = 10000
N_EDGES = 320000
D = 128
NPAD = 10240
CR = NPAD // D
NC = 2
NS = 16
EPT = N_EDGES // NS
K = 128
NCH = EPT // K
KT = EPT - NCH * K
ROWS_PT = NPAD // NS


def _sc_body(x_hbm, e0_hbm, e1_hbm, s_hbm, cnt_hbm,
             sb0, sb1, sb2, sb3, db0, db1, db2, db3, srct_v, dstt_v,
             rows0, rows1, cnt_v, iota_v, acc_sh, cnt_sh,
             si0, si1, si2, si3, sg0, sg1, ss0, ss1, sem_t):
    c = lax.axis_index("c")
    s = lax.axis_index("s")
    srcb = (sb0, sb1, sb2, sb3)
    dstb = (db0, db1, db2, db3)
    rows = (rows0, rows1)
    semi = (si0, si1, si2, si3)
    semg = (sg0, sg1)
    sems = (ss0, ss1)

    zero16 = jnp.zeros((16,), jnp.float32)

    def _zero_local():
        for rv in rows:
            def zrow(r, carry, rv=rv):
                def zcol(q, carry2):
                    rv[r, pl.ds(q * 16, 16)] = zero16
                    return carry2
                return lax.fori_loop(0, D // 16, zcol, carry)
            lax.fori_loop(0, K, zrow, 0)

        def zcnt(r, carry):
            def zcol(q, carry2):
                cnt_v[r, pl.ds(q * 16, 16)] = zero16
                return carry2
            return lax.fori_loop(0, D // 16, zcol, carry)
        lax.fori_loop(0, CR, zcnt, 0)

    def _zero_shared():
        def zslice(r, carry):
            pltpu.sync_copy(rows0, acc_sh.at[pl.ds(s * ROWS_PT + r * K, K)])
            return carry
        lax.fori_loop(0, ROWS_PT // K, zslice, 0)

        @pl.when(s < CR // 8)
        def _zero_cnt():
            pltpu.sync_copy(rows0.at[pl.ds(0, 8)], cnt_sh.at[pl.ds(s * 8, 8)])

    _zero_local()
    for q in range(5):
        iota_v[pl.ds(q * 16, 16)] = lax.iota(jnp.int32, 16) + q * 16
    _zero_shared()
    plsc.subcore_barrier()

    def _count(dst_ref, nvec):
        for q in range(nvec):
            dv = dst_ref[pl.ds(q * 16, 16)]
            occ, last = plsc.scan_count(dv)
            row = lax.shift_right_logical(dv, 7)
            col = lax.bitwise_and(dv, 127)
            plsc.addupdate_scatter(cnt_v, [row, col],
                                   occ.astype(jnp.float32), mask=last)

    if True:
        def _issue_idx(j, slot):
            @pl.when(c == 0)
            def _i0():
                pltpu.async_copy(e0_hbm.at[0, s, pl.ds(j * K, K)], srcb[slot], semi[slot])
                pltpu.async_copy(e0_hbm.at[1, s, pl.ds(j * K, K)], dstb[slot], semi[slot])

            @pl.when(c == 1)
            def _i1():
                pltpu.async_copy(e1_hbm.at[0, s, pl.ds(j * K, K)], srcb[slot], semi[slot])
                pltpu.async_copy(e1_hbm.at[1, s, pl.ds(j * K, K)], dstb[slot], semi[slot])

        def _wait_idx(j, slot):
            pltpu.make_async_copy(e0_hbm.at[0, s, pl.ds(j * K, K)], srcb[slot],
                                  semi[slot]).wait()
            pltpu.make_async_copy(e0_hbm.at[1, s, pl.ds(j * K, K)], dstb[slot],
                                  semi[slot]).wait()

        _issue_idx(0, 0)
        _issue_idx(1, 1)
        _wait_idx(0, 0)
        pltpu.async_copy(x_hbm.at[srcb[0]], rows[0], semg[0])

        def group(g, carry):
            for b in range(4):
                j = 4 * g + b
                rb = b % 2
                rb1 = (b + 1) % 2

                @pl.when(j < NCH - 1)
                def _advance():
                    _wait_idx(j + 1, (b + 1) % 4)

                    @pl.when(j >= 1)
                    def _drain_prev():
                        pltpu.make_async_copy(
                            rows[rb1], acc_sh.at[dstb[(b + 3) % 4]],
                            sems[rb1]).wait()

                    @pl.when(j < NCH - 2)
                    def _prefetch():
                        _issue_idx(j + 2, (b + 2) % 4)

                    pltpu.async_copy(x_hbm.at[srcb[(b + 1) % 4]], rows[rb1],
                                     semg[rb1])

                pltpu.make_async_copy(x_hbm.at[srcb[b]], rows[rb],
                                      semg[rb]).wait()
                pltpu.async_copy(rows[rb], acc_sh.at[dstb[b]], sems[rb], add=True)
                _count(dstb[b], K // 16)
            return carry
        lax.fori_loop(0, NCH // 4, group, 0)

        pltpu.make_async_copy(rows[0], acc_sh.at[dstb[2]], sems[0]).wait()
        pltpu.make_async_copy(rows[1], acc_sh.at[dstb[3]], sems[1]).wait()

        tb = NCH * K

        @pl.when(c == 0)
        def _t0():
            pltpu.sync_copy(e0_hbm.at[0, s, pl.ds(tb, KT)], srct_v)
            pltpu.sync_copy(e0_hbm.at[1, s, pl.ds(tb, KT)], dstt_v)

        @pl.when(c == 1)
        def _t1():
            pltpu.sync_copy(e1_hbm.at[0, s, pl.ds(tb, KT)], srct_v)
            pltpu.sync_copy(e1_hbm.at[1, s, pl.ds(tb, KT)], dstt_v)
        pltpu.async_copy(x_hbm.at[srct_v], rows0.at[pl.ds(0, KT)], sem_t).wait()
        pltpu.async_copy(rows0.at[pl.ds(0, KT)], acc_sh.at[dstt_v], sem_t,
                         add=True).wait()
        _count(dstt_v, KT // 16)

        pltpu.sync_copy(cnt_v, cnt_sh.at[iota_v], add=True)
        plsc.subcore_barrier()

        pltpu.sync_copy(acc_sh.at[pl.ds(s * ROWS_PT, ROWS_PT)],
                        s_hbm.at[c, pl.ds(s * ROWS_PT, ROWS_PT)])

        @pl.when(s < CR // 8)
        def _write_cnt():
            pltpu.sync_copy(cnt_sh.at[pl.ds(s * 8, 8)],
                            cnt_hbm.at[c, pl.ds(s * 8, 8)])


_sc_aggregate = functools.partial(
    pl.kernel,
    out_type=(jax.ShapeDtypeStruct((NC, NPAD, D), jnp.float32),
              jax.ShapeDtypeStruct((NC, CR, D), jnp.float32)),
    mesh=plsc.VectorSubcoreMesh(core_axis_name="c", subcore_axis_name="s",
                                num_cores=2),
    scratch_types=(
        [pltpu.VMEM((K,), jnp.int32)] * 4
        + [pltpu.VMEM((K,), jnp.int32)] * 4
        + [pltpu.VMEM((KT,), jnp.int32)] * 2
        + [pltpu.VMEM((K, D), jnp.float32)] * 2
        + [
            pltpu.VMEM((CR, D), jnp.float32),
            pltpu.VMEM((CR,), jnp.int32),
            pltpu.VMEM_SHARED((NPAD, D), jnp.float32),
            pltpu.VMEM_SHARED((CR, D), jnp.float32),
        ]
        + [pltpu.SemaphoreType.DMA] * 9
    ),
    compiler_params=pltpu.CompilerParams(needs_layout_passes=False),
)(_sc_body)


RB = 1000


def _tc_body(a0_ref, a1_ref, c0_ref, c1_ref, w0_ref, w1_ref, o_ref):
    s0 = a0_ref[0] / jnp.maximum(c0_ref[0], 1.0)
    s1 = a1_ref[0] / jnp.maximum(c1_ref[0], 1.0)
    dn = (((1,), (1,)), ((), ()))
    o_ref[...] = (
        lax.dot_general(s0, w0_ref[...], dn, preferred_element_type=jnp.float32)
        + lax.dot_general(s1, w1_ref[...], dn, preferred_element_type=jnp.float32)
    )


def kernel(x, edge_index_e0, edge_index_e1, W0, W1):
    e0 = edge_index_e0.reshape(2, NS, EPT)
    e1 = edge_index_e1.reshape(2, NS, EPT)

    acc, cnt = _sc_aggregate(x, e0, e1)
    cntc = cnt.reshape(NC, NPAD, 1)

    h = pl.pallas_call(
        _tc_body,
        grid=(N_NODES // RB,),
        in_specs=[
            pl.BlockSpec((1, RB, D), lambda i: (0, i, 0)),
            pl.BlockSpec((1, RB, D), lambda i: (1, i, 0)),
            pl.BlockSpec((1, RB, 1), lambda i: (0, i, 0)),
            pl.BlockSpec((1, RB, 1), lambda i: (1, i, 0)),
            pl.BlockSpec((D, D), lambda i: (0, 0)),
            pl.BlockSpec((D, D), lambda i: (0, 0)),
        ],
        out_specs=pl.BlockSpec((RB, D), lambda i: (i, 0)),
        out_shape=jax.ShapeDtypeStruct((N_NODES, D), jnp.float32),
    )(acc, acc, cntc, cntc, W0, W1)
    return h
